# baseline (speedup 1.0000x reference)
import jax
import jax.numpy as jnp
from jax.experimental import pallas as pl


def _head_body(idsT_ref, embT_ref, w_ref, o_ref):
    vpad = embT_ref.shape[1]
    bsz = idsT_ref.shape[2]
    ids = idsT_ref[0, 0, :]
    iota = jax.lax.broadcasted_iota(jnp.int32, (vpad, bsz), 0)
    onehot_t = (iota == ids[None, :]).astype(jnp.bfloat16)
    h_aug = jnp.dot(embT_ref[...], onehot_t, preferred_element_type=jnp.float32)
    o_ref[0] = jnp.dot(w_ref[...], h_aug, preferred_element_type=jnp.float32)


def _head_tc(ids_t3, emb_aug, w_aug):
    seq, _, bsz = ids_t3.shape
    vocab, e1 = w_aug.shape
    vpad = emb_aug.shape[1]
    return pl.pallas_call(
        _head_body,
        grid=(seq,),
        in_specs=[
            pl.BlockSpec((1, 1, bsz), lambda i: (i, 0, 0)),
            pl.BlockSpec((e1, vpad), lambda i: (0, 0)),
            pl.BlockSpec((vocab, e1), lambda i: (0, 0)),
        ],
        out_specs=pl.BlockSpec((1, vocab, bsz), lambda i: (i, 0, 0)),
        out_shape=jax.ShapeDtypeStruct((seq, vocab, bsz), jnp.float32),
    )(ids_t3, emb_aug, w_aug)


def kernel(input_ids, embed, W, b):
    bsz, seq = input_ids.shape
    vocab = W.shape[0]
    vpad = (vocab + 127) // 128 * 128
    ids_t3 = input_ids.astype(jnp.int32).T.reshape(seq, 1, bsz)
    emb_t = jnp.pad(embed.T, ((0, 0), (0, vpad - vocab)))
    ones_row = jnp.ones((1, vpad), jnp.float32)
    emb_aug = jnp.concatenate([emb_t, ones_row], axis=0).astype(jnp.bfloat16)
    w_aug = jnp.concatenate([W, b.reshape(vocab, 1)], axis=1)
    out_t = _head_tc(ids_t3, emb_aug, w_aug)
    return jnp.transpose(out_t, (2, 0, 1))


# speedup vs baseline: 1.0015x; 1.0015x over previous
import jax
import jax.numpy as jnp
from jax.experimental import pallas as pl


def _head_body(idsT_ref, embT_ref, w_ref, o_ref):
    vpad = embT_ref.shape[1]
    bsz = idsT_ref.shape[2]
    ids = idsT_ref[0, 0, :]
    iota = jax.lax.broadcasted_iota(jnp.int32, (vpad, bsz), 0)
    onehot_t = (iota == ids[None, :]).astype(jnp.float32)
    h_aug = jnp.dot(embT_ref[...], onehot_t, preferred_element_type=jnp.float32)
    o_ref[0] = jnp.dot(w_ref[...], h_aug, preferred_element_type=jnp.float32)


def _head_tc(ids_t3, emb_aug, w_aug):
    seq, _, bsz = ids_t3.shape
    vocab, e1 = w_aug.shape
    vpad = emb_aug.shape[1]
    return pl.pallas_call(
        _head_body,
        grid=(seq,),
        in_specs=[
            pl.BlockSpec((1, 1, bsz), lambda i: (i, 0, 0)),
            pl.BlockSpec((e1, vpad), lambda i: (0, 0)),
            pl.BlockSpec((vocab, e1), lambda i: (0, 0)),
        ],
        out_specs=pl.BlockSpec((1, vocab, bsz), lambda i: (i, 0, 0)),
        out_shape=jax.ShapeDtypeStruct((seq, vocab, bsz), jnp.float32),
    )(ids_t3, emb_aug, w_aug)


def kernel(input_ids, embed, W, b):
    bsz, seq = input_ids.shape
    vocab = W.shape[0]
    vpad = (vocab + 127) // 128 * 128
    ids_t3 = input_ids.astype(jnp.int32).T.reshape(seq, 1, bsz)
    emb_t = jnp.pad(embed.T, ((0, 0), (0, vpad - vocab)))
    ones_row = jnp.ones((1, vpad), jnp.float32)
    emb_aug = jnp.concatenate([emb_t, ones_row], axis=0)
    w_aug = jnp.concatenate([W, b.reshape(vocab, 1)], axis=1)
    out_t = _head_tc(ids_t3, emb_aug, w_aug)
    return jnp.transpose(out_t, (2, 0, 1))
